# baseline (speedup 1.0000x reference)
import functools

import jax
import jax.numpy as jnp
from jax import lax
from jax.experimental import pallas as pl
from jax.experimental.pallas import tpu as pltpu
from jax.experimental.pallas import tpu_sc as plsc

BATCH = 16384
HIST = 50
DIM = 16
NROWS = 100000

NC = 2
NS = 16
NW = NC * NS
ROWS_PER_SUB = BATCH // NW
RCHUNK = 16
NCHUNK = ROWS_PER_SUB // RCHUNK
ROWS_PER_TILE = NROWS // NS


@functools.partial(
    pl.kernel,
    out_type=jax.ShapeDtypeStruct((BATCH, HIST, DIM), jnp.float32),
    mesh=plsc.VectorSubcoreMesh(core_axis_name="c", subcore_axis_name="s"),
    scratch_types=[
        pltpu.VMEM_SHARED((NROWS, DIM), jnp.float32),
        pltpu.VMEM((RCHUNK, HIST), jnp.int32),
        pltpu.VMEM((RCHUNK, HIST), jnp.int32),
        pltpu.VMEM((RCHUNK, HIST, DIM), jnp.float32),
        pltpu.VMEM((RCHUNK, HIST, DIM), jnp.float32),
        pltpu.SemaphoreType.DMA,
        pltpu.SemaphoreType.DMA,
        pltpu.SemaphoreType.DMA,
        pltpu.SemaphoreType.DMA,
        pltpu.SemaphoreType.DMA,
        pltpu.SemaphoreType.DMA,
    ],
    compiler_params=pltpu.CompilerParams(use_tc_tiling_on_sc=False),
)
def _gather(idx_hbm, table_hbm, out_hbm, tbl_sh, i0, i1, r0, r1,
            si0, si1, sg0, sg1, so0, so1):
    sid = lax.axis_index("s")
    wid = sid * NC + lax.axis_index("c")
    base = wid * ROWS_PER_SUB
    idx_v = (i0, i1)
    rows_v = (r0, r1)
    si = (si0, si1)
    sg = (sg0, sg1)
    so = (so0, so1)

    def off(i):
        return base + i * RCHUNK

    def start_gather(b):
        return [
            pltpu.async_copy(
                tbl_sh.at[idx_v[b].at[j]], rows_v[b].at[j], sg[b])
            for j in range(RCHUNK)
        ]

    trow = sid * ROWS_PER_TILE
    stage = pltpu.async_copy(
        table_hbm.at[pl.ds(trow, ROWS_PER_TILE)],
        tbl_sh.at[pl.ds(trow, ROWS_PER_TILE)], sg0)
    idx_cp = [
        pltpu.async_copy(idx_hbm.at[pl.ds(off(0), RCHUNK)], i0, si0),
        pltpu.async_copy(idx_hbm.at[pl.ds(off(1), RCHUNK)], i1, si1),
    ]
    stage.wait()
    plsc.subcore_barrier()

    gat_cp = [None, None]
    out_cp = [None, None]

    idx_cp[0].wait()
    gat_cp[0] = start_gather(0)

    for i in range(NCHUNK):
        b = i % 2
        nb = (i + 1) % 2
        if i + 1 < NCHUNK:
            idx_cp[nb].wait()
            if out_cp[nb] is not None:
                out_cp[nb].wait()
            gat_cp[nb] = start_gather(nb)
        for cp in gat_cp[b]:
            cp.wait()
        out_cp[b] = pltpu.async_copy(
            rows_v[b], out_hbm.at[pl.ds(off(i), RCHUNK)], so[b])
        if i + 2 < NCHUNK:
            idx_cp[b] = pltpu.async_copy(
                idx_hbm.at[pl.ds(off(i + 2), RCHUNK)], idx_v[b], si[b])

    out_cp[0].wait()
    out_cp[1].wait()


def kernel(action_history, embedding_weight):
    out = _gather(action_history.astype(jnp.int32), embedding_weight)
    return out.reshape(BATCH, HIST * DIM)


# speedup vs baseline: 2.3498x; 2.3498x over previous
import functools

import jax
import jax.numpy as jnp
from jax import lax
from jax.experimental import pallas as pl
from jax.experimental.pallas import tpu as pltpu
from jax.experimental.pallas import tpu_sc as plsc

BATCH = 16384
HIST = 50
DIM = 16
NROWS = 100000

NC = 2
NS = 16
NW = NC * NS
BLK = BATCH // NW


@functools.partial(
    pl.kernel,
    out_type=jax.ShapeDtypeStruct((BATCH, HIST * DIM), jnp.float32),
    mesh=plsc.VectorSubcoreMesh(core_axis_name="c", subcore_axis_name="s"),
    scratch_types=[
        pltpu.VMEM((BLK,), jnp.int32),
        pltpu.VMEM((BLK,), jnp.int32),
        pltpu.VMEM((BLK, DIM), jnp.float32),
        pltpu.VMEM((BLK, DIM), jnp.float32),
        pltpu.SemaphoreType.DMA,
        pltpu.SemaphoreType.DMA,
        pltpu.SemaphoreType.DMA,
        pltpu.SemaphoreType.DMA,
        pltpu.SemaphoreType.DMA,
        pltpu.SemaphoreType.DMA,
    ],
    compiler_params=pltpu.CompilerParams(use_tc_tiling_on_sc=False),
)
def _gather(idxt_hbm, table_hbm, out_hbm, i0, i1, r0, r1,
            si0, si1, sg0, sg1, so0, so1):
    wid = lax.axis_index("s") * NC + lax.axis_index("c")
    base = wid * BLK
    idx_v = (i0, i1)
    rows_v = (r0, r1)
    si = (si0, si1)
    sg = (sg0, sg1)
    so = (so0, so1)

    def load_idx(l, b):
        return pltpu.async_copy(
            idxt_hbm.at[l, pl.ds(base, BLK)], idx_v[b], si[b])

    idx_cp = [load_idx(0, 0), load_idx(1, 1)]
    gat_cp = [None, None]
    out_cp = [None, None]

    idx_cp[0].wait()
    gat_cp[0] = pltpu.async_copy(table_hbm.at[i0], r0, sg[0])

    for l in range(HIST):
        b = l % 2
        nb = (l + 1) % 2
        if l + 1 < HIST:
            idx_cp[nb].wait()
            if out_cp[nb] is not None:
                out_cp[nb].wait()
            gat_cp[nb] = pltpu.async_copy(
                table_hbm.at[idx_v[nb]], rows_v[nb], sg[nb])
        gat_cp[b].wait()
        out_cp[b] = pltpu.async_copy(
            rows_v[b],
            out_hbm.at[pl.ds(base, BLK), pl.ds(l * DIM, DIM)], so[b])
        if l + 2 < HIST:
            idx_cp[b] = load_idx(l + 2, b)

    out_cp[0].wait()
    out_cp[1].wait()


def kernel(action_history, embedding_weight):
    idx_t = action_history.astype(jnp.int32).T
    return _gather(idx_t, embedding_weight)


# speedup vs baseline: 2.6315x; 1.1198x over previous
import functools

import jax
import jax.numpy as jnp
from jax import lax
from jax.experimental import pallas as pl
from jax.experimental.pallas import tpu as pltpu
from jax.experimental.pallas import tpu_sc as plsc

BATCH = 16384
HIST = 50
DIM = 16
TOTAL = BATCH * HIST
NROWS = 100000

NC = 2
NS = 16
NW = NC * NS
B_PER_W = TOTAL // NW
CHUNK = 512
NCHUNK = B_PER_W // CHUNK
NBUF = 3
ROWS_PER_TILE = NROWS // NS


@functools.partial(
    pl.kernel,
    out_type=jax.ShapeDtypeStruct((TOTAL, DIM), jnp.float32),
    mesh=plsc.VectorSubcoreMesh(core_axis_name="c", subcore_axis_name="s"),
    scratch_types=(
        [pltpu.VMEM_SHARED((NROWS, DIM), jnp.float32)]
        + [pltpu.VMEM((CHUNK,), jnp.int32) for _ in range(NBUF)]
        + [pltpu.VMEM((CHUNK, DIM), jnp.float32) for _ in range(NBUF)]
        + [pltpu.SemaphoreType.DMA for _ in range(3 * NBUF)]
    ),
    compiler_params=pltpu.CompilerParams(use_tc_tiling_on_sc=False),
)
def _gather(idx_hbm, table_hbm, out_hbm, tbl_sh, *bufs):
    idx_v = bufs[0:NBUF]
    rows_v = bufs[NBUF:2 * NBUF]
    si = bufs[2 * NBUF:3 * NBUF]
    sg = bufs[3 * NBUF:4 * NBUF]
    so = bufs[4 * NBUF:5 * NBUF]

    sid = lax.axis_index("s")
    wid = sid * NC + lax.axis_index("c")
    base = wid * B_PER_W

    def off(i):
        return base + i * CHUNK

    def load_idx(i, b):
        return pltpu.async_copy(
            idx_hbm.at[pl.ds(off(i), CHUNK)], idx_v[b], si[b])

    trow = sid * ROWS_PER_TILE
    stage = pltpu.async_copy(
        table_hbm.at[pl.ds(trow, ROWS_PER_TILE)],
        tbl_sh.at[pl.ds(trow, ROWS_PER_TILE)], sg[0])
    idx_cp = [load_idx(b, b) for b in range(NBUF)]
    stage.wait()
    plsc.subcore_barrier()

    gat_cp = [None] * NBUF
    out_cp = [None] * NBUF

    idx_cp[0].wait()
    gat_cp[0] = pltpu.async_copy(tbl_sh.at[idx_v[0]], rows_v[0], sg[0])

    for i in range(NCHUNK):
        b = i % NBUF
        nb = (i + 1) % NBUF
        if i + 1 < NCHUNK:
            idx_cp[nb].wait()
            if out_cp[nb] is not None:
                out_cp[nb].wait()
            gat_cp[nb] = pltpu.async_copy(
                tbl_sh.at[idx_v[nb]], rows_v[nb], sg[nb])
        gat_cp[b].wait()
        out_cp[b] = pltpu.async_copy(
            rows_v[b], out_hbm.at[pl.ds(off(i), CHUNK)], so[b])
        if i + NBUF < NCHUNK:
            idx_cp[b] = load_idx(i + NBUF, b)

    for b in range(NBUF):
        if out_cp[b] is not None:
            out_cp[b].wait()


def kernel(action_history, embedding_weight):
    idx = action_history.reshape(-1).astype(jnp.int32)
    out = _gather(idx, embedding_weight)
    return out.reshape(action_history.shape[0], -1)
